# initial kernel scaffold (unmeasured)
import jax
import jax.numpy as jnp
from jax import lax
from jax.experimental import pallas as pl
from jax.experimental.pallas import tpu as pltpu

N_DEV = 4
M, K, N = 4096, 4096, 8192
C = M // N_DEV


def kernel(x, w_mat):
    partial = jnp.dot(
        x, w_mat, preferred_element_type=jnp.float32
    ).astype(jnp.bfloat16)

    def body(
        p_hbm,
        out_hbm,
        comm,
        local,
        fstage,
        send_sems,
        recv_sems,
        local_sem,
        out_sem,
    ):
        my = lax.axis_index("i")
        left = lax.rem(my + N_DEV - 1, N_DEV)
        right = lax.rem(my + 1, N_DEV)

        barrier_sem = pltpu.get_barrier_semaphore()
        for nbr in (left, right):
            pl.semaphore_signal(
                barrier_sem,
                inc=1,
                device_id=(nbr,),
                device_id_type=pl.DeviceIdType.MESH,
            )
        pl.semaphore_wait(barrier_sem, 2)

        cp = pltpu.make_async_copy(
            p_hbm.at[pl.ds(my * C, C)], comm.at[0], local_sem
        )
        cp.start()
        cp.wait()

        def store_chunk(slot, c_idx):
            fstage[...] = jnp.maximum(comm[slot].astype(jnp.float32), 0.0)
            ocp = pltpu.make_async_copy(
                fstage, out_hbm.at[pl.ds(c_idx * C, C)], out_sem
            )
            ocp.start()
            ocp.wait()

        def hop(h):
            send_slot = h % 2
            recv_slot = (h + 1) % 2
            rdma = pltpu.make_async_remote_copy(
                src_ref=comm.at[send_slot],
                dst_ref=comm.at[recv_slot],
                send_sem=send_sems.at[send_slot],
                recv_sem=recv_sems.at[recv_slot],
                device_id=(right,),
                device_id_type=pl.DeviceIdType.MESH,
            )
            rdma.start()
            return rdma, recv_slot

        for s in range(N_DEV - 1):
            rdma, recv_slot = hop(s)
            c_in = lax.rem(my - s - 1 + 2 * N_DEV, N_DEV)
            lcp = pltpu.make_async_copy(
                p_hbm.at[pl.ds(c_in * C, C)], local, local_sem
            )
            lcp.start()
            lcp.wait()
            rdma.wait()
            comm[recv_slot] = (
                comm[recv_slot].astype(jnp.float32)
                + local[...].astype(jnp.float32)
            ).astype(jnp.bfloat16)

        store_chunk(1, lax.rem(my + 1, N_DEV))

        for t in range(N_DEV - 1):
            rdma, recv_slot = hop(N_DEV - 1 + t)
            rdma.wait()
            c_idx = lax.rem(my - t + 2 * N_DEV, N_DEV)
            store_chunk(recv_slot, c_idx)

    return pl.pallas_call(
        body,
        out_shape=jax.ShapeDtypeStruct((M, N), jnp.float32),
        in_specs=[pl.BlockSpec(memory_space=pltpu.ANY)],
        out_specs=pl.BlockSpec(memory_space=pltpu.ANY),
        scratch_shapes=[
            pltpu.VMEM((2, C, N), jnp.bfloat16),
            pltpu.VMEM((C, N), jnp.bfloat16),
            pltpu.VMEM((C, N), jnp.float32),
            pltpu.SemaphoreType.DMA((2,)),
            pltpu.SemaphoreType.DMA((2,)),
            pltpu.SemaphoreType.DMA,
            pltpu.SemaphoreType.DMA,
        ],
        compiler_params=pltpu.CompilerParams(collective_id=0),
    )(partial)


# baseline (device time: 1349230 ns/iter reference)
import jax
import jax.numpy as jnp
from jax import lax
from jax.experimental import pallas as pl
from jax.experimental.pallas import tpu as pltpu

N_DEV = 4
M, K, N = 4096, 4096, 8192
C = M // N_DEV
NH = N // 2
Q = C // 4


def kernel(x, w_mat):
    partial = jnp.dot(
        x, w_mat, preferred_element_type=jnp.float32
    ).astype(jnp.bfloat16)

    def body(
        p_hbm,
        out_hbm,
        comm,
        local,
        fstage,
        send_sems,
        recv_sems,
        local_sem,
        out_sem,
    ):
        my = lax.axis_index("i")
        left = lax.rem(my + N_DEV - 1, N_DEV)
        right = lax.rem(my + 1, N_DEV)

        barrier_sem = pltpu.get_barrier_semaphore()
        for nbr in (left, right):
            pl.semaphore_signal(
                barrier_sem,
                inc=1,
                device_id=(nbr,),
                device_id_type=pl.DeviceIdType.MESH,
            )
        pl.semaphore_wait(barrier_sem, 2)

        for half in range(2):
            col = half * NH

            def hop(h):
                send_slot = h % 2
                recv_slot = (h + 1) % 2
                rdma = pltpu.make_async_remote_copy(
                    src_ref=comm.at[send_slot],
                    dst_ref=comm.at[recv_slot],
                    send_sem=send_sems.at[half, send_slot],
                    recv_sem=recv_sems.at[half, recv_slot],
                    device_id=(right,),
                    device_id_type=pl.DeviceIdType.MESH,
                )
                rdma.start()
                return rdma, recv_slot

            def store_chunk(slot, c_idx):
                for q in range(4):
                    fstage[...] = jnp.maximum(
                        comm[slot, pl.ds(q * Q, Q), :].astype(jnp.float32),
                        0.0,
                    )
                    ocp = pltpu.make_async_copy(
                        fstage,
                        out_hbm.at[pl.ds(c_idx * C + q * Q, Q), pl.ds(col, NH)],
                        out_sem,
                    )
                    ocp.start()
                    ocp.wait()

            cp = pltpu.make_async_copy(
                p_hbm.at[pl.ds(my * C, C), pl.ds(col, NH)],
                comm.at[0],
                local_sem,
            )
            cp.start()
            cp.wait()

            for s in range(N_DEV - 1):
                rdma, recv_slot = hop(s)
                c_in = lax.rem(my - s - 1 + 2 * N_DEV, N_DEV)
                lcp = pltpu.make_async_copy(
                    p_hbm.at[pl.ds(c_in * C, C), pl.ds(col, NH)],
                    local,
                    local_sem,
                )
                lcp.start()
                lcp.wait()
                rdma.wait()
                for q in range(4):
                    sl = pl.ds(q * Q, Q)
                    comm[recv_slot, sl, :] = (
                        comm[recv_slot, sl, :] + local[sl, :]
                    )

            store_chunk(1, lax.rem(my + 1, N_DEV))

            for t in range(N_DEV - 1):
                rdma, recv_slot = hop(N_DEV - 1 + t)
                rdma.wait()
                c_idx = lax.rem(my - t + 2 * N_DEV, N_DEV)
                store_chunk(recv_slot, c_idx)

    return pl.pallas_call(
        body,
        out_shape=jax.ShapeDtypeStruct((M, N), jnp.float32),
        in_specs=[pl.BlockSpec(memory_space=pl.ANY)],
        out_specs=pl.BlockSpec(memory_space=pl.ANY),
        scratch_shapes=[
            pltpu.VMEM((2, C, NH), jnp.bfloat16),
            pltpu.VMEM((C, NH), jnp.bfloat16),
            pltpu.VMEM((Q, NH), jnp.float32),
            pltpu.SemaphoreType.DMA((2, 2)),
            pltpu.SemaphoreType.DMA((2, 2)),
            pltpu.SemaphoreType.DMA,
            pltpu.SemaphoreType.DMA,
        ],
        compiler_params=pltpu.CompilerParams(collective_id=0),
    )(partial)


# device time: 758400 ns/iter; 1.7790x vs baseline; 1.7790x over previous
import jax
import jax.numpy as jnp
from jax import lax
from jax.experimental import pallas as pl
from jax.experimental.pallas import tpu as pltpu

N_DEV = 4
M, K, N = 4096, 4096, 8192
C = M // N_DEV
CH = C // 2
NH = N // 2
Q = C // 4


def kernel(x, w_mat):
    partial = jnp.dot(
        x, w_mat, preferred_element_type=jnp.float32
    ).astype(jnp.bfloat16)

    def body(
        p_hbm,
        out_hbm,
        comm_r,
        comm_l,
        local_r,
        local_l,
        fstage,
        send_sems,
        recv_sems,
        local_sems,
        out_sem,
    ):
        my = lax.axis_index("i")
        left = lax.rem(my + N_DEV - 1, N_DEV)
        right = lax.rem(my + 1, N_DEV)

        comms = (comm_r, comm_l)
        locals_ = (local_r, local_l)
        targets = (right, left)
        cols = (0, NH)

        barrier_sem = pltpu.get_barrier_semaphore()
        for nbr in (left, right):
            pl.semaphore_signal(
                barrier_sem,
                inc=1,
                device_id=(nbr,),
                device_id_type=pl.DeviceIdType.MESH,
            )
        pl.semaphore_wait(barrier_sem, 2)

        def hop(d, h):
            send_slot = h % 2
            recv_slot = (h + 1) % 2
            rdma = pltpu.make_async_remote_copy(
                src_ref=comms[d].at[send_slot],
                dst_ref=comms[d].at[recv_slot],
                send_sem=send_sems.at[d, send_slot],
                recv_sem=recv_sems.at[d, recv_slot],
                device_id=(targets[d],),
                device_id_type=pl.DeviceIdType.MESH,
            )
            rdma.start()
            return rdma, recv_slot

        def store_piece(d, slot, c_idx, po):
            for q in range(CH // Q):
                fstage[...] = jnp.maximum(
                    comms[d][slot, pl.ds(q * Q, Q), :].astype(jnp.float32),
                    0.0,
                )
                ocp = pltpu.make_async_copy(
                    fstage,
                    out_hbm.at[
                        pl.ds(c_idx * C + po + q * Q, Q), pl.ds(cols[d], NH)
                    ],
                    out_sem,
                )
                ocp.start()
                ocp.wait()

        for p in range(2):
            po = p * CH

            for d in range(2):
                pltpu.make_async_copy(
                    p_hbm.at[pl.ds(my * C + po, CH), pl.ds(cols[d], NH)],
                    comms[d].at[0],
                    local_sems.at[d],
                ).start()
            for d in range(2):
                pltpu.make_async_copy(
                    p_hbm.at[pl.ds(0, CH), pl.ds(cols[d], NH)],
                    comms[d].at[0],
                    local_sems.at[d],
                ).wait()

            for s in range(N_DEV - 1):
                rdmas = []
                for d in range(2):
                    rdma, recv_slot = hop(d, s)
                    rdmas.append((rdma, recv_slot))
                    c_in = lax.rem(
                        my + (s + 1) * (1 if d else -1) + 2 * N_DEV, N_DEV
                    )
                    pltpu.make_async_copy(
                        p_hbm.at[
                            pl.ds(c_in * C + po, CH), pl.ds(cols[d], NH)
                        ],
                        locals_[d],
                        local_sems.at[d],
                    ).start()
                for d in range(2):
                    rdma, recv_slot = rdmas[d]
                    rdma.wait()
                    pltpu.make_async_copy(
                        p_hbm.at[pl.ds(0, CH), pl.ds(cols[d], NH)],
                        locals_[d],
                        local_sems.at[d],
                    ).wait()
                    for q in range(CH // Q):
                        sl = pl.ds(q * Q, Q)
                        comms[d][recv_slot, sl, :] = (
                            comms[d][recv_slot, sl, :] + locals_[d][sl, :]
                        )

            pending = [
                (0, 1, lax.rem(my + 1, N_DEV)),
                (1, 1, lax.rem(my + N_DEV - 1, N_DEV)),
            ]

            for t in range(N_DEV - 1):
                rdmas = []
                for d in range(2):
                    rdma, recv_slot = hop(d, N_DEV - 1 + t)
                    rdmas.append((rdma, recv_slot))
                for d, slot, c_idx in pending:
                    store_piece(d, slot, c_idx, po)
                pending = []
                for d in range(2):
                    rdma, recv_slot = rdmas[d]
                    rdma.wait()
                    c_idx = lax.rem(
                        my + t * (1 if d else -1) + 2 * N_DEV, N_DEV
                    )
                    pending.append((d, recv_slot, c_idx))
            for d, slot, c_idx in pending:
                store_piece(d, slot, c_idx, po)

    return pl.pallas_call(
        body,
        out_shape=jax.ShapeDtypeStruct((M, N), jnp.float32),
        in_specs=[pl.BlockSpec(memory_space=pl.ANY)],
        out_specs=pl.BlockSpec(memory_space=pl.ANY),
        scratch_shapes=[
            pltpu.VMEM((2, CH, NH), jnp.bfloat16),
            pltpu.VMEM((2, CH, NH), jnp.bfloat16),
            pltpu.VMEM((CH, NH), jnp.bfloat16),
            pltpu.VMEM((CH, NH), jnp.bfloat16),
            pltpu.VMEM((Q, NH), jnp.float32),
            pltpu.SemaphoreType.DMA((2, 2)),
            pltpu.SemaphoreType.DMA((2, 2)),
            pltpu.SemaphoreType.DMA((2,)),
            pltpu.SemaphoreType.DMA,
        ],
        compiler_params=pltpu.CompilerParams(collective_id=0),
    )(partial)


# device time: 716888 ns/iter; 1.8821x vs baseline; 1.0579x over previous
import jax
import jax.numpy as jnp
from jax import lax
from jax.experimental import pallas as pl
from jax.experimental.pallas import tpu as pltpu

N_DEV = 4
M, K, N = 4096, 4096, 8192
C = M // N_DEV
CH = 256
NH = N // 2
N_HOPS = 6
N_PASS = 2


def kernel(x, w_mat):
    partial = jnp.dot(
        x, w_mat, preferred_element_type=jnp.float32
    ).astype(jnp.bfloat16)

    def body(
        p_hbm,
        out_hbm,
        c_r0,
        c_r1,
        c_l0,
        c_l1,
        l_r0,
        l_r1,
        l_l0,
        l_l1,
        fstage,
        send_sems_0,
        send_sems_1,
        recv_sems_0,
        recv_sems_1,
        local_sems,
        out_sem,
    ):
        send_sems = (send_sems_0, send_sems_1)
        recv_sems = (recv_sems_0, recv_sems_1)
        my = lax.axis_index("i")
        left = lax.rem(my + N_DEV - 1, N_DEV)
        right = lax.rem(my + 1, N_DEV)

        comms = ((c_r0, c_r1), (c_l0, c_l1))
        locals_ = ((l_r0, l_r1), (l_l0, l_l1))
        targets = (right, left)
        cols = (0, NH)

        def sgn(d):
            return 1 if d else -1

        barrier_sem = pltpu.get_barrier_semaphore()
        for nbr in (left, right):
            pl.semaphore_signal(
                barrier_sem,
                inc=1,
                device_id=(nbr,),
                device_id_type=pl.DeviceIdType.MESH,
            )
        pl.semaphore_wait(barrier_sem, 2)

        def piece_off(st, p):
            return (p * 2 + st) * CH

        def store_piece(d, st, slot, c_idx, po):
            fstage[...] = jnp.maximum(
                comms[d][st][slot].astype(jnp.float32), 0.0
            )
            ocp = pltpu.make_async_copy(
                fstage,
                out_hbm.at[pl.ds(c_idx * C + po, CH), pl.ds(cols[d], NH)],
                out_sem,
            )
            ocp.start()
            ocp.wait()

        def own_load(st, p):
            po = piece_off(st, p)
            for d in range(2):
                pltpu.make_async_copy(
                    p_hbm.at[pl.ds(my * C + po, CH), pl.ds(cols[d], NH)],
                    comms[d][st].at[0],
                    local_sems.at[d, st],
                ).start()
            for d in range(2):
                pltpu.make_async_copy(
                    p_hbm.at[pl.ds(0, CH), pl.ds(cols[d], NH)],
                    comms[d][st].at[0],
                    local_sems.at[d, st],
                ).wait()

        def start_hops(st, hg):
            h = hg % N_HOPS
            po = piece_off(st, hg // N_HOPS)
            ss, rs = hg % 2, (hg + 1) % 2
            out = []
            for d in range(2):
                rdma = pltpu.make_async_remote_copy(
                    src_ref=comms[d][st].at[ss],
                    dst_ref=comms[d][st].at[rs],
                    send_sem=send_sems[st].at[d, ss],
                    recv_sem=recv_sems[st].at[d, rs],
                    device_id=(targets[d],),
                    device_id_type=pl.DeviceIdType.MESH,
                )
                rdma.start()
                out.append((rdma, rs))
            if h <= 2:
                for d in range(2):
                    c_in = lax.rem(
                        my + sgn(d) * (h + 1) + 2 * N_DEV, N_DEV
                    )
                    pltpu.make_async_copy(
                        p_hbm.at[
                            pl.ds(c_in * C + po, CH), pl.ds(cols[d], NH)
                        ],
                        locals_[d][st],
                        local_sems.at[d, st],
                    ).start()
            if h == 3:
                for d in range(2):
                    c_own = lax.rem(
                        my + sgn(d) * (N_DEV - 1) + 2 * N_DEV, N_DEV
                    )
                    store_piece(d, st, 1, c_own, po)
            return out

        def process(st, hg, rd):
            h = hg % N_HOPS
            po = piece_off(st, hg // N_HOPS)
            for d in range(2):
                rdma, rs = rd[d]
                rdma.wait()
                if h <= 2:
                    pltpu.make_async_copy(
                        p_hbm.at[pl.ds(0, CH), pl.ds(cols[d], NH)],
                        locals_[d][st],
                        local_sems.at[d, st],
                    ).wait()
                    comms[d][st][rs] = comms[d][st][rs] + locals_[d][st][...]
                else:
                    c_idx = lax.rem(
                        my + sgn(d) * (h - 3) + 2 * N_DEV, N_DEV
                    )
                    store_piece(d, st, rs, c_idx, po)

        own_load(0, 0)
        rd0 = start_hops(0, 0)
        own_load(1, 0)
        rd1 = start_hops(1, 0)
        total = N_HOPS * N_PASS
        for hg in range(total):
            process(0, hg, rd0)
            if hg + 1 < total:
                if (hg + 1) % N_HOPS == 0:
                    own_load(0, (hg + 1) // N_HOPS)
                rd0 = start_hops(0, hg + 1)
            process(1, hg, rd1)
            if hg + 1 < total:
                if (hg + 1) % N_HOPS == 0:
                    own_load(1, (hg + 1) // N_HOPS)
                rd1 = start_hops(1, hg + 1)

    comm_shape = pltpu.VMEM((2, CH, NH), jnp.bfloat16)
    local_shape = pltpu.VMEM((CH, NH), jnp.bfloat16)
    return pl.pallas_call(
        body,
        out_shape=jax.ShapeDtypeStruct((M, N), jnp.float32),
        in_specs=[pl.BlockSpec(memory_space=pl.ANY)],
        out_specs=pl.BlockSpec(memory_space=pl.ANY),
        scratch_shapes=[
            comm_shape,
            comm_shape,
            comm_shape,
            comm_shape,
            local_shape,
            local_shape,
            local_shape,
            local_shape,
            pltpu.VMEM((CH, NH), jnp.float32),
            pltpu.SemaphoreType.DMA((2, 2)),
            pltpu.SemaphoreType.DMA((2, 2)),
            pltpu.SemaphoreType.DMA((2, 2)),
            pltpu.SemaphoreType.DMA((2, 2)),
            pltpu.SemaphoreType.DMA((2, 2)),
            pltpu.SemaphoreType.DMA,
        ],
        compiler_params=pltpu.CompilerParams(collective_id=0),
    )(partial)
